# Initial kernel scaffold; baseline (speedup 1.0000x reference)
import jax
import jax.numpy as jnp
from jax.experimental import pallas as pl


def kernel(x, edge_index, batch, Wl0, bl0, Wr0, gamma0, beta0, Wl1, bl1, Wr1, gamma1, beta1, Wl2, bl2, Wr2, gamma2, beta2, W1, b1, W2, b2):
    raise NotImplementedError("write your pallas kernel here")


# speedup vs baseline: 4.4223x; 4.4223x over previous
import functools

import jax
import jax.numpy as jnp
from jax import lax
from jax.experimental import pallas as pl
from jax.experimental.pallas import tpu as pltpu
from jax.experimental.pallas import tpu_sc as plsc

N = 10000
E = 320000
G = 256
H = 128
DH = 64
NC = 2
NS = 16
NW = NC * NS
CH = 80
EPW = E // NW
NCH_W = EPW // CH
SLAB = (N // NS) // 8 * 8
TAIL = N - NS * SLAB
DEGW = 16

_mesh = plsc.VectorSubcoreMesh(
    core_axis_name="c", subcore_axis_name="s", num_cores=NC, num_subcores=NS
)


def _each_slab(s, fn):
    fn(s * SLAB, SLAB)

    @pl.when(s == NS - 1)
    def _():
        fn(NS * SLAB, TAIL)


@functools.partial(
    pl.kernel,
    out_type=jax.ShapeDtypeStruct((NC, N, H), jnp.float32),
    mesh=_mesh,
    scratch_types=[
        pltpu.VMEM((1, CH), jnp.int32),
        pltpu.VMEM((1, CH), jnp.int32),
        pltpu.VMEM((CH, H), jnp.float32),
        pltpu.VMEM_SHARED((N, H), jnp.float32),
        pltpu.SemaphoreType.DMA,
    ],
)
def _agg_kernel(h_hbm, src_hbm, dst_hbm, zeros_h, accp,
                idx_s, idx_d, rows, acc_sp, sem):
    c = lax.axis_index("c")
    s = lax.axis_index("s")
    wid = s * NC + c

    _each_slab(s, lambda start, size: pltpu.sync_copy(
        zeros_h.at[pl.ds(start, size)], acc_sp.at[pl.ds(start, size)]))
    plsc.subcore_barrier()

    ebase = wid * EPW

    def body(g, carry):
        eb = ebase + g * CH
        pltpu.sync_copy(src_hbm.at[pl.ds(eb, CH)], idx_s.at[0])
        pltpu.async_copy(h_hbm.at[idx_s.at[0]], rows, sem).wait()
        pltpu.sync_copy(dst_hbm.at[pl.ds(eb, CH)], idx_d.at[0])
        pltpu.sync_copy(rows, acc_sp.at[idx_d.at[0]], add=True)
        return carry

    lax.fori_loop(0, NCH_W, body, 0)
    plsc.subcore_barrier()

    _each_slab(s, lambda start, size: pltpu.sync_copy(
        acc_sp.at[pl.ds(start, size)], accp.at[c, pl.ds(start, size)]))


@functools.partial(
    pl.kernel,
    out_type=jax.ShapeDtypeStruct((NC, N, H), jnp.float32),
    mesh=_mesh,
    scratch_types=[
        pltpu.VMEM((1, CH), jnp.int32),
        pltpu.VMEM((CH, H), jnp.float32),
        pltpu.VMEM_SHARED((N, H), jnp.float32),
    ],
)
def _deg_kernel(dst_hbm, zeros_d, ones_hbm, degp, idx_d, ones_v, deg_sp):
    c = lax.axis_index("c")
    s = lax.axis_index("s")
    wid = s * NC + c

    _each_slab(s, lambda start, size: pltpu.sync_copy(
        zeros_d.at[pl.ds(start, size)], deg_sp.at[pl.ds(start, size)]))
    pltpu.sync_copy(ones_hbm, ones_v)
    plsc.subcore_barrier()

    ebase = wid * EPW

    def body(g, carry):
        eb = ebase + g * CH
        pltpu.sync_copy(dst_hbm.at[pl.ds(eb, CH)], idx_d.at[0])
        pltpu.sync_copy(ones_v, deg_sp.at[idx_d.at[0]], add=True)
        return carry

    lax.fori_loop(0, NCH_W, body, 0)
    plsc.subcore_barrier()

    _each_slab(s, lambda start, size: pltpu.sync_copy(
        deg_sp.at[pl.ds(start, size)], degp.at[c, pl.ds(start, size)]))


def _layer_pre(accp_ref, degp_ref, h_ref, wlT_ref, bl_ref, wrT_ref,
               gamma_ref, beta_ref):
    acc = accp_ref[0] + accp_ref[1]
    deg = degp_ref[0, :, 0:1] + degp_ref[1, :, 0:1]
    mean = acc / jnp.maximum(deg, 1.0)
    pre = (jnp.dot(mean, wlT_ref[...], preferred_element_type=jnp.float32)
           + bl_ref[...]
           + jnp.dot(h_ref[...], wrT_ref[...],
                     preferred_element_type=jnp.float32))
    m = jnp.mean(pre, axis=0, keepdims=True)
    v = jnp.mean((pre - m) ** 2, axis=0, keepdims=True)
    hn = (pre - m) * lax.rsqrt(v + 1e-5) * gamma_ref[...] + beta_ref[...]
    return jnp.maximum(hn, 0.0)


def _dense_body(accp_ref, degp_ref, h_ref, wlT_ref, bl_ref, wrT_ref,
                gamma_ref, beta_ref, out_ref):
    out_ref[...] = _layer_pre(accp_ref, degp_ref, h_ref, wlT_ref, bl_ref,
                              wrT_ref, gamma_ref, beta_ref)


def _final_body(accp_ref, degp_ref, h_ref, wlT_ref, bl_ref, wrT_ref,
                gamma_ref, beta_ref, batch_ref, w1T_ref, b1_ref, w2T_ref,
                b2_ref, out_ref):
    h3 = _layer_pre(accp_ref, degp_ref, h_ref, wlT_ref, bl_ref, wrT_ref,
                    gamma_ref, beta_ref)
    onehot = (lax.broadcasted_iota(jnp.int32, (G, N), 0)
              == batch_ref[...]).astype(jnp.float32)
    psum = jnp.dot(onehot, h3, preferred_element_type=jnp.float32)
    cnt = jnp.sum(onehot, axis=1, keepdims=True)
    pooled = psum / jnp.maximum(cnt, 1.0)
    hid = jnp.maximum(
        jnp.dot(pooled, w1T_ref[...], preferred_element_type=jnp.float32)
        + b1_ref[...], 0.0)
    out_ref[...] = (jnp.dot(hid, w2T_ref[...],
                            preferred_element_type=jnp.float32)
                    + b2_ref[...])


_dense_call = pl.pallas_call(
    _dense_body, out_shape=jax.ShapeDtypeStruct((N, H), jnp.float32))

_final_call = pl.pallas_call(
    _final_body, out_shape=jax.ShapeDtypeStruct((G, 1), jnp.float32))


def kernel(x, edge_index, batch, Wl0, bl0, Wr0, gamma0, beta0,
           Wl1, bl1, Wr1, gamma1, beta1, Wl2, bl2, Wr2, gamma2, beta2,
           W1, b1, W2, b2):
    src = edge_index[0]
    dst = edge_index[1]
    zeros_h = jnp.zeros((N, H), jnp.float32)
    ones_hbm = jnp.ones((CH, H), jnp.float32)
    batch2d = batch.reshape(1, N)

    degp = _deg_kernel(dst, zeros_h, ones_hbm)
    accp0 = _agg_kernel(x, src, dst, zeros_h)
    h1 = _dense_call(accp0, degp, x, Wl0.T, bl0.reshape(1, H), Wr0.T,
                     gamma0.reshape(1, H), beta0.reshape(1, H))
    accp1 = _agg_kernel(h1, src, dst, zeros_h)
    h2 = _dense_call(accp1, degp, h1, Wl1.T, bl1.reshape(1, H), Wr1.T,
                     gamma1.reshape(1, H), beta1.reshape(1, H))
    accp2 = _agg_kernel(h2, src, dst, zeros_h)
    out = _final_call(accp2, degp, h2, Wl2.T, bl2.reshape(1, H), Wr2.T,
                      gamma2.reshape(1, H), beta2.reshape(1, H),
                      batch2d, W1.T, b1.reshape(1, DH), W2.T,
                      b2.reshape(1, 1))
    return out


# speedup vs baseline: 7.7134x; 1.7442x over previous
import functools

import jax
import jax.numpy as jnp
from jax import lax
from jax.experimental import pallas as pl
from jax.experimental.pallas import tpu as pltpu
from jax.experimental.pallas import tpu_sc as plsc

N = 10000
E = 320000
G = 256
H = 128
DH = 64
NC = 2
NS = 16
NW = NC * NS
CH = 80
EPW = E // NW
NCH_W = EPW // CH
SLAB = (N // NS) // 8 * 8
TAIL = N - NS * SLAB
DEGW = 16

_mesh = plsc.VectorSubcoreMesh(
    core_axis_name="c", subcore_axis_name="s", num_cores=NC, num_subcores=NS
)


def _each_slab(s, fn):
    fn(s * SLAB, SLAB)

    @pl.when(s == NS - 1)
    def _():
        fn(NS * SLAB, TAIL)


@functools.partial(
    pl.kernel,
    out_type=jax.ShapeDtypeStruct((NC, N, H), jnp.float32),
    mesh=_mesh,
    scratch_types=[
        pltpu.VMEM((1, CH), jnp.int32),
        pltpu.VMEM((1, CH), jnp.int32),
        pltpu.VMEM((NCH_W, CH), jnp.int32),
        pltpu.VMEM((CH, H), jnp.float32),
        pltpu.VMEM((CH, H), jnp.float32),
        pltpu.VMEM_SHARED((N, H), jnp.float32),
        pltpu.SemaphoreType.DMA,
        pltpu.SemaphoreType.DMA,
    ],
)
def _agg_kernel(h_hbm, src3d, dst3d, zeros_h, accp,
                idx_s0, idx_s1, idx_d_all, rows0, rows1, acc_sp,
                sem0, sem1):
    c = lax.axis_index("c")
    s = lax.axis_index("s")
    wid = s * NC + c

    _each_slab(s, lambda start, size: pltpu.sync_copy(
        zeros_h.at[pl.ds(start, size)], acc_sp.at[pl.ds(start, size)]))
    pltpu.sync_copy(dst3d.at[wid], idx_d_all)
    plsc.subcore_barrier()

    def _gstart(g, ibuf, rbuf, sem):
        pltpu.sync_copy(src3d.at[wid, g], ibuf.at[0])
        pltpu.async_copy(h_hbm.at[ibuf.at[0]], rbuf, sem)

    def _gwait(ibuf, rbuf, sem):
        pltpu.make_async_copy(h_hbm.at[ibuf.at[0]], rbuf, sem).wait()

    def _scatter(g, rbuf):
        pltpu.sync_copy(rbuf, acc_sp.at[idx_d_all.at[g]], add=True)

    _gstart(0, idx_s0, rows0, sem0)

    def pair(p, carry):
        g = 2 * p
        _gstart(g + 1, idx_s1, rows1, sem1)
        _gwait(idx_s0, rows0, sem0)
        _scatter(g, rows0)
        _gstart(g + 2, idx_s0, rows0, sem0)
        _gwait(idx_s1, rows1, sem1)
        _scatter(g + 1, rows1)
        return carry

    lax.fori_loop(0, (NCH_W - 1) // 2, pair, 0)
    _gwait(idx_s0, rows0, sem0)
    _scatter(NCH_W - 1, rows0)
    plsc.subcore_barrier()

    _each_slab(s, lambda start, size: pltpu.sync_copy(
        acc_sp.at[pl.ds(start, size)], accp.at[c, pl.ds(start, size)]))


@functools.partial(
    pl.kernel,
    out_type=jax.ShapeDtypeStruct((NC, N, H), jnp.float32),
    mesh=_mesh,
    scratch_types=[
        pltpu.VMEM((1, CH), jnp.int32),
        pltpu.VMEM((CH, H), jnp.float32),
        pltpu.VMEM_SHARED((N, H), jnp.float32),
    ],
)
def _deg_kernel(dst_hbm, zeros_d, ones_hbm, degp, idx_d, ones_v, deg_sp):
    c = lax.axis_index("c")
    s = lax.axis_index("s")
    wid = s * NC + c

    _each_slab(s, lambda start, size: pltpu.sync_copy(
        zeros_d.at[pl.ds(start, size)], deg_sp.at[pl.ds(start, size)]))
    pltpu.sync_copy(ones_hbm, ones_v)
    plsc.subcore_barrier()

    ebase = wid * EPW

    def body(g, carry):
        eb = ebase + g * CH
        pltpu.sync_copy(dst_hbm.at[pl.ds(eb, CH)], idx_d.at[0])
        pltpu.sync_copy(ones_v, deg_sp.at[idx_d.at[0]], add=True)
        return carry

    lax.fori_loop(0, NCH_W, body, 0)
    plsc.subcore_barrier()

    _each_slab(s, lambda start, size: pltpu.sync_copy(
        deg_sp.at[pl.ds(start, size)], degp.at[c, pl.ds(start, size)]))


def _layer_pre(accp_ref, degp_ref, h_ref, wlT_ref, bl_ref, wrT_ref,
               gamma_ref, beta_ref):
    acc = accp_ref[0] + accp_ref[1]
    deg = degp_ref[0, :, 0:1] + degp_ref[1, :, 0:1]
    mean = acc / jnp.maximum(deg, 1.0)
    pre = (jnp.dot(mean, wlT_ref[...], preferred_element_type=jnp.float32)
           + bl_ref[...]
           + jnp.dot(h_ref[...], wrT_ref[...],
                     preferred_element_type=jnp.float32))
    m = jnp.mean(pre, axis=0, keepdims=True)
    v = jnp.mean((pre - m) ** 2, axis=0, keepdims=True)
    hn = (pre - m) * lax.rsqrt(v + 1e-5) * gamma_ref[...] + beta_ref[...]
    return jnp.maximum(hn, 0.0)


def _dense_body(accp_ref, degp_ref, h_ref, wlT_ref, bl_ref, wrT_ref,
                gamma_ref, beta_ref, out_ref):
    out_ref[...] = _layer_pre(accp_ref, degp_ref, h_ref, wlT_ref, bl_ref,
                              wrT_ref, gamma_ref, beta_ref)


def _final_body(accp_ref, degp_ref, h_ref, wlT_ref, bl_ref, wrT_ref,
                gamma_ref, beta_ref, batch_ref, w1T_ref, b1_ref, w2T_ref,
                b2_ref, out_ref):
    h3 = _layer_pre(accp_ref, degp_ref, h_ref, wlT_ref, bl_ref, wrT_ref,
                    gamma_ref, beta_ref)
    onehot = (lax.broadcasted_iota(jnp.int32, (G, N), 0)
              == batch_ref[...]).astype(jnp.float32)
    psum = jnp.dot(onehot, h3, preferred_element_type=jnp.float32)
    cnt = jnp.sum(onehot, axis=1, keepdims=True)
    pooled = psum / jnp.maximum(cnt, 1.0)
    hid = jnp.maximum(
        jnp.dot(pooled, w1T_ref[...], preferred_element_type=jnp.float32)
        + b1_ref[...], 0.0)
    out_ref[...] = (jnp.dot(hid, w2T_ref[...],
                            preferred_element_type=jnp.float32)
                    + b2_ref[...])


_dense_call = pl.pallas_call(
    _dense_body, out_shape=jax.ShapeDtypeStruct((N, H), jnp.float32))

_final_call = pl.pallas_call(
    _final_body, out_shape=jax.ShapeDtypeStruct((G, 1), jnp.float32))


def kernel(x, edge_index, batch, Wl0, bl0, Wr0, gamma0, beta0,
           Wl1, bl1, Wr1, gamma1, beta1, Wl2, bl2, Wr2, gamma2, beta2,
           W1, b1, W2, b2):
    src = edge_index[0]
    dst = edge_index[1]
    src3d = src.reshape(NW, NCH_W, CH)
    dst3d = dst.reshape(NW, NCH_W, CH)
    zeros_h = jnp.zeros((N, H), jnp.float32)
    ones_hbm = jnp.ones((CH, H), jnp.float32)
    batch2d = batch.reshape(1, N)

    degp = _deg_kernel(dst, zeros_h, ones_hbm)
    accp0 = _agg_kernel(x, src3d, dst3d, zeros_h)
    h1 = _dense_call(accp0, degp, x, Wl0.T, bl0.reshape(1, H), Wr0.T,
                     gamma0.reshape(1, H), beta0.reshape(1, H))
    accp1 = _agg_kernel(h1, src3d, dst3d, zeros_h)
    h2 = _dense_call(accp1, degp, h1, Wl1.T, bl1.reshape(1, H), Wr1.T,
                     gamma1.reshape(1, H), beta1.reshape(1, H))
    accp2 = _agg_kernel(h2, src3d, dst3d, zeros_h)
    out = _final_call(accp2, degp, h2, Wl2.T, bl2.reshape(1, H), Wr2.T,
                      gamma2.reshape(1, H), beta2.reshape(1, H),
                      batch2d, W1.T, b1.reshape(1, DH), W2.T,
                      b2.reshape(1, 1))
    return out
